# baseline (speedup 1.0000x reference)
import functools

import jax
import jax.numpy as jnp
from jax import lax
from jax.experimental import pallas as pl
from jax.experimental.pallas import tpu as pltpu
from jax.experimental.pallas import tpu_sc as plsc

M = 30000
NV = 20000
NN = M + NV
DIM = 64
B = 4096
NEG = 40
REG = 1e-4

NW = 32
BPW = B // NW
CHUNK = 128
NCH = NEG + 2
SGB = 16
NSG = BPW // SGB
SGC = SGB * NEG // CHUNK
L = 16
PQ = BPW * NEG // 8

_IL = plsc.PackFormat.INTERLEAVED


def _sc_factory():
    mesh = plsc.VectorSubcoreMesh(core_axis_name="c", subcore_axis_name="s")

    @functools.partial(
        pl.kernel,
        out_type=(
            jax.ShapeDtypeStruct((NW, PQ, 128), jnp.float32),
            jax.ShapeDtypeStruct((NW, 17, 128), jnp.float32),
        ),
        mesh=mesh,
        scratch_types=[
            pltpu.VMEM((NCH, CHUNK), jnp.int32),
            pltpu.VMEM((2, SGB * NEG, DIM), jnp.bfloat16),
            pltpu.VMEM((CHUNK, DIM), jnp.bfloat16),
            pltpu.VMEM((CHUNK, DIM), jnp.bfloat16),
            pltpu.VMEM((2, SGB * NEG // 8, 128), jnp.float32),
            pltpu.VMEM((17, 128), jnp.float32),
            pltpu.SemaphoreType.DMA((2,)),
            pltpu.SemaphoreType.DMA((2,)),
            pltpu.SemaphoreType.DMA,
        ],
        compiler_params=pltpu.CompilerParams(use_tc_tiling_on_sc=False,
                                             needs_layout_passes=False),
    )
    def sc_bpr(nidx_hbm, uidx_hbm, vidx_hbm, z_hbm, np_hbm, uv_hbm,
               idx_v, rows_v, u_v, v_v, psum_v, uvp_v, gsem, wsem, usem):
        wid = lax.axis_index("s") * 2 + lax.axis_index("c")
        pltpu.sync_copy(nidx_hbm.at[wid], idx_v.at[pl.ds(0, NEG)])
        pltpu.sync_copy(uidx_hbm.at[wid], idx_v.at[NEG])
        pltpu.sync_copy(vidx_hbm.at[wid], idx_v.at[NEG + 1])

        def start_sg_gathers(sg, slot):
            for k in range(SGC):
                pltpu.async_copy(
                    z_hbm.at[idx_v.at[sg * SGC + k]],
                    rows_v.at[slot, pl.ds(k * CHUNK, CHUNK)],
                    gsem.at[slot])

        def wait_sg_gathers(slot):
            for _ in range(SGC):
                pltpu.make_async_copy(
                    z_hbm.at[idx_v.at[0]],
                    rows_v.at[slot, pl.ds(0, CHUNK)],
                    gsem.at[slot]).wait()

        pltpu.async_copy(z_hbm.at[idx_v.at[NEG]], u_v, usem)
        pltpu.async_copy(z_hbm.at[idx_v.at[NEG + 1]], v_v, usem)
        start_sg_gathers(0, 0)
        start_sg_gathers(1, 1)
        pltpu.make_async_copy(z_hbm.at[idx_v.at[0]], u_v, usem).wait()
        pltpu.make_async_copy(z_hbm.at[idx_v.at[0]], v_v, usem).wait()

        zero = jnp.zeros((L,), jnp.float32)
        for k in range(8):
            uvp_v[16, pl.ds(k * L, L)] = zero

        def fold(x32):
            lo, hi = plsc.unpack(x32, format=_IL,
                                 preferred_element_type=jnp.float32)
            return lo, hi

        def uv_body(b, racc):
            r0, r1 = racc
            u0 = u_v[b, pl.ds(0, 2 * L)]
            u1 = u_v[b, pl.ds(2 * L, 2 * L)]
            v0 = v_v[b, pl.ds(0, 2 * L)]
            v1 = v_v[b, pl.ds(2 * L, 2 * L)]
            plo, phi = fold(u0 * v0 + u1 * v1)
            uvp_v[b // 8, pl.ds((b % 8) * L, L)] = plo + phi
            slo, shi = fold(u0 * u0 + u1 * u1)
            tlo, thi = fold(v0 * v0 + v1 * v1)
            return (r0 + slo + tlo, r1 + shi + thi)

        racc = lax.fori_loop(0, BPW, uv_body, (zero, zero), unroll=False)

        for sg in range(NSG):
            slot = sg % 2
            if sg >= 2:
                pltpu.make_async_copy(
                    psum_v.at[slot], np_hbm.at[wid, pl.ds(0, PQ // NSG)],
                    wsem.at[slot]).wait()
            wait_sg_gathers(slot)

            def b_body(bl, racc, _slot=slot, _sg=sg):
                b = _sg * SGB + bl
                u0 = u_v[b, pl.ds(0, 2 * L)]
                u1 = u_v[b, pl.ds(2 * L, 2 * L)]

                def j_body(j, racc2):
                    r0, r1 = racc2
                    row = bl * NEG + j
                    n0 = rows_v[_slot, row, pl.ds(0, 2 * L)]
                    n1 = rows_v[_slot, row, pl.ds(2 * L, 2 * L)]
                    plo, phi = fold(n0 * u0 + n1 * u1)
                    psum_v[_slot, row // 8, pl.ds((row % 8) * L, L)] = (
                        plo + phi)
                    slo, shi = fold(n0 * n0 + n1 * n1)
                    return (r0 + slo, r1 + shi)

                return lax.fori_loop(0, NEG, j_body, racc, unroll=2)

            racc = lax.fori_loop(0, SGB, b_body, racc, unroll=False)
            pltpu.async_copy(
                psum_v.at[slot],
                np_hbm.at[wid, pl.ds(sg * (PQ // NSG), PQ // NSG)],
                wsem.at[slot])
            if sg + 2 < NSG:
                start_sg_gathers(sg + 2, slot)

        uvp_v[16, pl.ds(0, L)] = racc[0] + racc[1]
        for slot in range(2):
            pltpu.make_async_copy(
                psum_v.at[slot], np_hbm.at[wid, pl.ds(0, PQ // NSG)],
                wsem.at[slot]).wait()
        pltpu.sync_copy(uvp_v, uv_hbm.at[wid])

    return sc_bpr


_sc_bpr = _sc_factory()


def _mmt(a, b):
    return jax.lax.dot_general(a, b, (((0,), (1,)), ((), ())),
                               preferred_element_type=jnp.float32)


def _tc_reduce_body(np_ref, uv_ref, w_ref, out_ref):
    i = pl.program_id(0)
    f32 = jnp.float32

    def iota(shape, d):
        return lax.broadcasted_iota(jnp.int32, shape, d)

    S = (iota((128, 8), 0) // L == iota((128, 8), 1)).astype(f32)
    X = np_ref[0]
    negT = _mmt(S, X)
    U = uv_ref[0]
    pos = jax.lax.dot_general(U[:16], S, (((1,), (0,)), ((), ())),
                              preferred_element_type=f32)
    reg = jnp.sum(U[16, :L])
    wv = w_ref[0]
    cpq = (1.5 - 0.5 * jnp.sign(wv)) * pos
    E1 = (iota((128, 16), 0) // 8 == iota((128, 16), 1)).astype(f32)
    Msel = (iota((128, 8), 0) % 8 == iota((128, 8), 1)).astype(f32)
    cpcol = jax.lax.dot_general(
        jax.lax.dot_general(E1, cpq, (((1,), (0,)), ((), ())),
                            preferred_element_type=f32) * Msel,
        jnp.ones((8, 1), f32), (((1,), (0,)), ((), ())),
        preferred_element_type=f32)
    M3 = (iota((640, 128), 0) // (NEG // 8) == iota((640, 128), 1)).astype(f32)
    cpT = _mmt(cpcol, M3)
    sT = cpT - negT
    spT = jnp.maximum(-sT, 0.0) + jnp.log1p(jnp.exp(-jnp.abs(sT)))
    partial = jnp.sum(spT) + REG * reg

    @pl.when(i == 0)
    def _():
        out_ref[...] = jnp.zeros_like(out_ref)

    out_ref[...] += partial.reshape(1, 1)


def kernel(u, v, n, w, E, E2, z, edge_index, W0, b0, W1, b1,
           attn_W, attn_b, q_W):
    del E, E2, edge_index, W0, b0, W1, b1, attn_W, attn_b, q_W
    u = u.astype(jnp.int32)
    v = v.astype(jnp.int32)
    n = n.astype(jnp.int32)
    zb = z.astype(jnp.bfloat16)
    np_out, uv_out = _sc_bpr(
        n.reshape(NW, NEG, CHUNK), u.reshape(NW, CHUNK),
        v.reshape(NW, CHUNK), zb)

    out = pl.pallas_call(
        _tc_reduce_body,
        grid=(NW,),
        in_specs=[
            pl.BlockSpec((1, PQ, 128), lambda i: (i, 0, 0)),
            pl.BlockSpec((1, 17, 128), lambda i: (i, 0, 0)),
            pl.BlockSpec((1, 16, 8), lambda i: (i, 0, 0)),
        ],
        out_specs=pl.BlockSpec((1, 1), lambda i: (0, 0)),
        out_shape=jax.ShapeDtypeStruct((1, 1), jnp.float32),
    )(np_out, uv_out, w.reshape(NW, 16, 8))
    return out[0, 0]
